# baseline (speedup 1.0000x reference)
import functools

import jax
import jax.numpy as jnp
from jax import lax
from jax.experimental import pallas as pl
from jax.experimental.pallas import tpu as pltpu
from jax.experimental.pallas import tpu_sc as plsc

B = 16384
D = 128
NC = 2
NS = 16
NW = NC * NS
CH = 128
NBUF = 4
NCHUNK = 4
BC = B // NCHUNK
BPW = BC // NW
NCH = BPW // CH


def _sc_gather_body(chunk, uemb_hbm, iemb_hbm, u_hbm, i_hbm, eu_hbm, ei_hbm,
                    uidx_v, iidx_v, bufs, gsem, wsem):
  wid = lax.axis_index("s") * NC + lax.axis_index("c")
  base = wid * BPW

  pltpu.sync_copy(u_hbm.at[pl.ds(chunk * BC + base, BPW)], uidx_v)
  pltpu.sync_copy(i_hbm.at[pl.ds(chunk * BC + base, BPW)], iidx_v)

  def stage(s):
    if s < NCH:
      return (uemb_hbm.at[uidx_v.at[pl.ds(s * CH, CH)]],
              eu_hbm.at[pl.ds(base + s * CH, CH)])
    c = s - NCH
    return (iemb_hbm.at[iidx_v.at[pl.ds(c * CH, CH)]],
            ei_hbm.at[pl.ds(base + c * CH, CH)])

  nst = 2 * NCH
  gathers = [None] * nst
  writes = [None] * nst
  src0, _ = stage(0)
  gathers[0] = pltpu.async_copy(src0, bufs.at[0], gsem)
  for s in range(nst):
    buf = bufs.at[s % NBUF]
    if s + 1 < nst:
      if s + 1 >= NBUF:
        writes[s + 1 - NBUF].wait()
      src, _ = stage(s + 1)
      gathers[s + 1] = pltpu.async_copy(src, bufs.at[(s + 1) % NBUF], gsem)
    gathers[s].wait()
    _, dst = stage(s)
    writes[s] = pltpu.async_copy(buf, dst, wsem)
  for s in range(max(0, nst - NBUF), nst):
    writes[s].wait()


def _sc_gather(user_emb, item_emb, u, i, chunk):
  mesh = plsc.VectorSubcoreMesh(core_axis_name="c", subcore_axis_name="s")
  out = jax.ShapeDtypeStruct((BC, D), jnp.float32)
  return pl.kernel(
      functools.partial(_sc_gather_body, chunk),
      out_type=(out, out),
      mesh=mesh,
      scratch_types=[
          pltpu.VMEM((BPW,), jnp.int32),
          pltpu.VMEM((BPW,), jnp.int32),
          pltpu.VMEM((NBUF, CH, D), jnp.float32),
          pltpu.SemaphoreType.DMA,
          pltpu.SemaphoreType.DMA,
      ],
  )(user_emb, item_emb, u, i)


BLK = 4096


def _mlp_body(eu, ei, w1, b1, w2, b2, w3, b3c, w4c, b4, out):
  w1v = w1[...]
  h = jnp.dot(eu[...], w1v[:D], preferred_element_type=jnp.float32)
  h = h + jnp.dot(ei[...], w1v[D:], preferred_element_type=jnp.float32)
  h = jnp.maximum(h + b1[...], 0.0)
  h = jnp.maximum(jnp.dot(h, w2[...], preferred_element_type=jnp.float32)
                  + b2[...], 0.0)
  h3t = lax.dot_general(w3[...], h, (((0,), (1,)), ((), ())),
                        preferred_element_type=jnp.float32)
  h3t = jnp.maximum(h3t + b3c[...], 0.0)
  o = jnp.sum(h3t * w4c[...], axis=0) + b4[0]
  out[...] = 1.0 / (1.0 + jnp.exp(-o))


def _mlp(eu, ei, W1, b1, W2, b2, W3, b3, W4, b4):
  whole = lambda shape: pl.BlockSpec(shape, lambda n: (0,) * len(shape))
  return pl.pallas_call(
      _mlp_body,
      grid=(BC // BLK,),
      in_specs=[
          pl.BlockSpec((BLK, D), lambda n: (n, 0)),
          pl.BlockSpec((BLK, D), lambda n: (n, 0)),
          whole((2 * D, 128)),
          whole((128,)),
          whole((128, 64)),
          whole((64,)),
          whole((64, 16)),
          whole((16, 1)),
          whole((16, 1)),
          whole((1,)),
      ],
      out_specs=pl.BlockSpec((BLK,), lambda n: (n,)),
      out_shape=jax.ShapeDtypeStruct((BC,), jnp.float32),
  )(eu, ei, W1, b1, W2, b2, W3, b3.reshape(16, 1), W4, b4)


def kernel(u, i, user_emb, item_emb, W1, b1, W2, b2, W3, b3, W4, b4):
  u = u.astype(jnp.int32)
  i = i.astype(jnp.int32)
  outs = []
  for k in range(NCHUNK):
    eu, ei = _sc_gather(user_emb, item_emb, u, i, k)
    outs.append(_mlp(eu, ei, W1, b1, W2, b2, W3, b3, W4, b4))
  return jnp.concatenate(outs) if NCHUNK > 1 else outs[0]


# speedup vs baseline: 1.2341x; 1.2341x over previous
import functools

import jax
import jax.numpy as jnp
from jax import lax
from jax.experimental import pallas as pl
from jax.experimental.pallas import tpu as pltpu
from jax.experimental.pallas import tpu_sc as plsc

B = 16384
D = 128
NC = 2
NS = 16
NW = NC * NS
CH = 128
NBUF = 4
NCHUNK = 2
BC = B // NCHUNK
BPW = BC // NW
NCH = BPW // CH


def _sc_gather_body(chunk, uemb_hbm, iemb_hbm, u_hbm, i_hbm, eu_hbm, ei_hbm,
                    uidx_v, iidx_v, bufs, gsem, wsem):
  wid = lax.axis_index("s") * NC + lax.axis_index("c")
  base = wid * BPW

  cu = pltpu.async_copy(u_hbm.at[pl.ds(chunk * BC + base, BPW)], uidx_v, gsem)
  ci = pltpu.async_copy(i_hbm.at[pl.ds(chunk * BC + base, BPW)], iidx_v, wsem)

  def stage(s):
    if s < NCH:
      return (uemb_hbm.at[uidx_v.at[pl.ds(s * CH, CH)]],
              eu_hbm.at[pl.ds(base + s * CH, CH)])
    c = s - NCH
    return (iemb_hbm.at[iidx_v.at[pl.ds(c * CH, CH)]],
            ei_hbm.at[pl.ds(base + c * CH, CH)])

  nst = 2 * NCH
  gathers = [None] * nst
  writes = [None] * nst
  cu.wait()
  for s in range(NCH):
    gathers[s] = pltpu.async_copy(stage(s)[0], bufs.at[s % NBUF], gsem)
  ci.wait()
  for s in range(NCH, nst):
    gathers[s] = pltpu.async_copy(stage(s)[0], bufs.at[s % NBUF], gsem)
  for s in range(nst):
    gathers[s].wait()
    writes[s] = pltpu.async_copy(bufs.at[s % NBUF], stage(s)[1], wsem)
  for s in range(nst):
    writes[s].wait()


def _sc_gather(user_emb, item_emb, u, i, chunk):
  mesh = plsc.VectorSubcoreMesh(core_axis_name="c", subcore_axis_name="s")
  out = jax.ShapeDtypeStruct((BC, D), jnp.float32)
  return pl.kernel(
      functools.partial(_sc_gather_body, chunk),
      out_type=(out, out),
      mesh=mesh,
      scratch_types=[
          pltpu.VMEM((BPW,), jnp.int32),
          pltpu.VMEM((BPW,), jnp.int32),
          pltpu.VMEM((NBUF, CH, D), jnp.float32),
          pltpu.SemaphoreType.DMA,
          pltpu.SemaphoreType.DMA,
      ],
  )(user_emb, item_emb, u, i)


BLK = 4096


def _mlp_body(eu, ei, w1, b1, w2, b2, w3, b3c, w4c, b4, out):
  w1v = w1[...]
  h = jnp.dot(eu[...], w1v[:D], preferred_element_type=jnp.float32)
  h = h + jnp.dot(ei[...], w1v[D:], preferred_element_type=jnp.float32)
  h = jnp.maximum(h + b1[...], 0.0)
  h = jnp.maximum(jnp.dot(h, w2[...], preferred_element_type=jnp.float32)
                  + b2[...], 0.0)
  h3t = lax.dot_general(w3[...], h, (((0,), (1,)), ((), ())),
                        preferred_element_type=jnp.float32)
  h3t = jnp.maximum(h3t + b3c[...], 0.0)
  o = jnp.sum(h3t * w4c[...], axis=0) + b4[0]
  out[...] = 1.0 / (1.0 + jnp.exp(-o))


def _mlp(eu, ei, W1, b1, W2, b2, W3, b3, W4, b4):
  whole = lambda shape: pl.BlockSpec(shape, lambda n: (0,) * len(shape))
  return pl.pallas_call(
      _mlp_body,
      grid=(BC // BLK,),
      in_specs=[
          pl.BlockSpec((BLK, D), lambda n: (n, 0)),
          pl.BlockSpec((BLK, D), lambda n: (n, 0)),
          whole((2 * D, 128)),
          whole((128,)),
          whole((128, 64)),
          whole((64,)),
          whole((64, 16)),
          whole((16, 1)),
          whole((16, 1)),
          whole((1,)),
      ],
      out_specs=pl.BlockSpec((BLK,), lambda n: (n,)),
      out_shape=jax.ShapeDtypeStruct((BC,), jnp.float32),
  )(eu, ei, W1, b1, W2, b2, W3, b3.reshape(16, 1), W4, b4)


def kernel(u, i, user_emb, item_emb, W1, b1, W2, b2, W3, b3, W4, b4):
  u = u.astype(jnp.int32)
  i = i.astype(jnp.int32)
  outs = []
  for k in range(NCHUNK):
    eu, ei = _sc_gather(user_emb, item_emb, u, i, k)
    outs.append(_mlp(eu, ei, W1, b1, W2, b2, W3, b3, W4, b4))
  return jnp.concatenate(outs) if NCHUNK > 1 else outs[0]


# speedup vs baseline: 1.2739x; 1.0323x over previous
import functools

import jax
import jax.numpy as jnp
from jax import lax
from jax.experimental import pallas as pl
from jax.experimental.pallas import tpu as pltpu
from jax.experimental.pallas import tpu_sc as plsc

B = 16384
D = 128
NC = 2
NS = 16
NW = NC * NS
CH = 128
NBUF = 4
NCHUNK = 2
BC = B // NCHUNK
BPW = BC // NW
NCH = BPW // CH


def _sc_gather_body(chunk, uemb_hbm, iemb_hbm, u_hbm, i_hbm, eu_hbm, ei_hbm,
                    uidx_v, iidx_v, bufs, gsem, wsem):
  wid = lax.axis_index("s") * NC + lax.axis_index("c")
  base = wid * BPW

  cu = pltpu.async_copy(u_hbm.at[pl.ds(chunk * BC + base, BPW)], uidx_v, gsem)
  ci = pltpu.async_copy(i_hbm.at[pl.ds(chunk * BC + base, BPW)], iidx_v, wsem)

  def stage(s):
    if s < NCH:
      return (uemb_hbm.at[uidx_v.at[pl.ds(s * CH, CH)]],
              eu_hbm.at[pl.ds(base + s * CH, CH)])
    c = s - NCH
    return (iemb_hbm.at[iidx_v.at[pl.ds(c * CH, CH)]],
            ei_hbm.at[pl.ds(base + c * CH, CH)])

  nst = 2 * NCH
  gathers = [None] * nst
  writes = [None] * nst
  cu.wait()
  for s in range(NCH):
    gathers[s] = pltpu.async_copy(stage(s)[0], bufs.at[s % NBUF], gsem)
  ci.wait()
  for s in range(NCH, nst):
    gathers[s] = pltpu.async_copy(stage(s)[0], bufs.at[s % NBUF], gsem)
  for s in range(nst):
    gathers[s].wait()
    writes[s] = pltpu.async_copy(bufs.at[s % NBUF], stage(s)[1], wsem)
  for s in range(nst):
    writes[s].wait()


def _sc_gather(user_emb, item_emb, u, i, chunk):
  mesh = plsc.VectorSubcoreMesh(core_axis_name="c", subcore_axis_name="s")
  out = jax.ShapeDtypeStruct((BC, D), jnp.float32)
  return pl.kernel(
      functools.partial(_sc_gather_body, chunk),
      out_type=(out, out),
      mesh=mesh,
      scratch_types=[
          pltpu.VMEM((BPW,), jnp.int32),
          pltpu.VMEM((BPW,), jnp.int32),
          pltpu.VMEM((NBUF, CH, D), jnp.float32),
          pltpu.SemaphoreType.DMA,
          pltpu.SemaphoreType.DMA,
      ],
  )(user_emb, item_emb, u, i)


BLK = 4096


def _mlp_body(eu, ei, w1, b1, w2, b2, w3, b3c, w4c, b4, acc, out):
  del acc
  w1v = w1[...]
  h = jnp.dot(eu[...], w1v[:D], preferred_element_type=jnp.float32)
  h = h + jnp.dot(ei[...], w1v[D:], preferred_element_type=jnp.float32)
  h = jnp.maximum(h + b1[...], 0.0)
  h = jnp.maximum(jnp.dot(h, w2[...], preferred_element_type=jnp.float32)
                  + b2[...], 0.0)
  h3t = lax.dot_general(w3[...], h, (((0,), (1,)), ((), ())),
                        preferred_element_type=jnp.float32)
  h3t = jnp.maximum(h3t + b3c[...], 0.0)
  o = jnp.sum(h3t * w4c[...], axis=0) + b4[0]
  out[...] = 1.0 / (1.0 + jnp.exp(-o))


def _mlp(eu, ei, W1, b1, W2, b2, W3, b3, W4, b4, acc, chunk):
  whole = lambda shape: pl.BlockSpec(shape, lambda n: (0,) * len(shape))
  goff = chunk * (BC // BLK)
  return pl.pallas_call(
      _mlp_body,
      grid=(BC // BLK,),
      in_specs=[
          pl.BlockSpec((BLK, D), lambda n: (n, 0)),
          pl.BlockSpec((BLK, D), lambda n: (n, 0)),
          whole((2 * D, 128)),
          whole((128,)),
          whole((128, 64)),
          whole((64,)),
          whole((64, 16)),
          whole((16, 1)),
          whole((16, 1)),
          whole((1,)),
          pl.BlockSpec(memory_space=pl.ANY),
      ],
      out_specs=pl.BlockSpec((BLK,), lambda n: (goff + n,)),
      out_shape=jax.ShapeDtypeStruct((B,), jnp.float32),
      input_output_aliases={10: 0},
  )(eu, ei, W1, b1, W2, b2, W3, b3.reshape(16, 1), W4, b4, acc)


def kernel(u, i, user_emb, item_emb, W1, b1, W2, b2, W3, b3, W4, b4):
  u = u.astype(jnp.int32)
  i = i.astype(jnp.int32)
  acc = jnp.zeros((B,), jnp.float32)
  for k in range(NCHUNK):
    eu, ei = _sc_gather(user_emb, item_emb, u, i, k)
    acc = _mlp(eu, ei, W1, b1, W2, b2, W3, b3, W4, b4, acc, k)
  return acc
